# baseline (speedup 1.0000x reference)
import jax
import jax.numpy as jnp
from jax.experimental import pallas as pl
from jax.experimental.pallas import tpu as pltpu

INPUT_DIM = 2048
NUM_EXPERTS = 64
BLOCK_T = 2048


def _epilogue(logits, disp_ref, probs_ref, sel_ref, w_ref):
    eidf = jax.lax.broadcasted_iota(jnp.int32, logits.shape, 1).astype(jnp.float32)
    riota = 63.0 - eidf
    a1 = jnp.max(logits, axis=1, keepdims=True)
    i1f = 63.0 - jnp.max(jnp.where(logits == a1, riota, -1.0), axis=1, keepdims=True)
    masked = jnp.where(eidf == i1f, -jnp.inf, logits)
    a2 = jnp.max(masked, axis=1, keepdims=True)
    i2f = 63.0 - jnp.max(jnp.where(masked == a2, riota, -1.0), axis=1, keepdims=True)

    e = jnp.exp(logits - a1)
    probs_ref[...] = e / jnp.sum(e, axis=1, keepdims=True)

    r = jnp.exp(a2 - a1)
    w1 = 1.0 / (1.0 + r)
    w2 = r * w1
    disp_ref[...] = jnp.where(
        eidf == i1f, w1, jnp.where(eidf == i2f, w2, jnp.zeros_like(logits))
    )
    sel_ref[...] = jnp.concatenate([i1f, i2f], axis=1).astype(jnp.int32).T
    w_ref[...] = jnp.concatenate([w1, w2], axis=1).T


def _router_body(x_ref, wt_ref, disp_ref, probs_ref, sel_ref, w_ref, scr_a, scr_b):
    i = pl.program_id(0)
    even = i % 2 == 0

    @pl.when(even)
    def _():
        scr_a[...] = jnp.dot(
            x_ref[...], wt_ref[...], preferred_element_type=jnp.float32
        )
        _epilogue(scr_b[...], disp_ref, probs_ref, sel_ref, w_ref)

    @pl.when(jnp.logical_not(even))
    def _():
        scr_b[...] = jnp.dot(
            x_ref[...], wt_ref[...], preferred_element_type=jnp.float32
        )
        _epilogue(scr_a[...], disp_ref, probs_ref, sel_ref, w_ref)


@jax.jit
def kernel(x, W):
    B, S, D = x.shape
    T = B * S
    N = T // BLOCK_T
    x2 = x.reshape(T, D)
    wt = W.T
    disp, probs, sel_t, wts_t = pl.pallas_call(
        _router_body,
        grid=(N + 1,),
        in_specs=[
            pl.BlockSpec((BLOCK_T, D), lambda i: (jnp.minimum(i, N - 1), 0)),
            pl.BlockSpec((D, NUM_EXPERTS), lambda i: (0, 0)),
        ],
        out_specs=[
            pl.BlockSpec((BLOCK_T, NUM_EXPERTS), lambda i: (jnp.maximum(i - 1, 0), 0)),
            pl.BlockSpec((BLOCK_T, NUM_EXPERTS), lambda i: (jnp.maximum(i - 1, 0), 0)),
            pl.BlockSpec((2, BLOCK_T), lambda i: (0, jnp.maximum(i - 1, 0))),
            pl.BlockSpec((2, BLOCK_T), lambda i: (0, jnp.maximum(i - 1, 0))),
        ],
        out_shape=[
            jax.ShapeDtypeStruct((T, NUM_EXPERTS), jnp.float32),
            jax.ShapeDtypeStruct((T, NUM_EXPERTS), jnp.float32),
            jax.ShapeDtypeStruct((2, T), jnp.int32),
            jax.ShapeDtypeStruct((2, T), jnp.float32),
        ],
        scratch_shapes=[
            pltpu.VMEM((BLOCK_T, NUM_EXPERTS), jnp.float32),
            pltpu.VMEM((BLOCK_T, NUM_EXPERTS), jnp.float32),
        ],
    )(x2, wt)
    return (
        disp.reshape(B, S, NUM_EXPERTS),
        probs.reshape(B, S, NUM_EXPERTS),
        sel_t.T.reshape(B, S, 2),
        wts_t.T.reshape(B, S, 2),
    )


# speedup vs baseline: 1.0092x; 1.0092x over previous
import jax
import jax.numpy as jnp
from jax.experimental import pallas as pl
from jax.experimental.pallas import tpu as pltpu

INPUT_DIM = 2048
NUM_EXPERTS = 64
BLOCK_T = 2048


def _epilogue(logits, disp_ref, probs_ref, sel_ref, w_ref):
    eidf = jax.lax.broadcasted_iota(jnp.int32, logits.shape, 1).astype(jnp.float32)
    riota = 63.0 - eidf
    a1 = jnp.max(logits, axis=1, keepdims=True)
    i1f = 63.0 - jnp.max(jnp.where(logits == a1, riota, -1.0), axis=1, keepdims=True)
    masked = jnp.where(eidf == i1f, -jnp.inf, logits)
    a2 = jnp.max(masked, axis=1, keepdims=True)
    i2f = 63.0 - jnp.max(jnp.where(masked == a2, riota, -1.0), axis=1, keepdims=True)

    e = jnp.exp(logits - a1)
    probs_ref[...] = e / jnp.sum(e, axis=1, keepdims=True)

    r = jnp.exp(a2 - a1)
    w1 = 1.0 / (1.0 + r)
    w2 = r * w1
    disp_ref[...] = jnp.where(
        eidf == i1f, w1, jnp.where(eidf == i2f, w2, jnp.zeros_like(logits))
    )
    sel_ref[...] = jnp.concatenate([i1f, i2f], axis=1).astype(jnp.int32).T
    w_ref[...] = jnp.concatenate([w1, w2], axis=1).T


def _router_body(x_ref, wt_ref, disp_ref, probs_ref, sel_ref, w_ref):
    logits = jnp.dot(x_ref[...], wt_ref[...], preferred_element_type=jnp.float32)
    _epilogue(logits, disp_ref, probs_ref, sel_ref, w_ref)


@jax.jit
def kernel(x, W):
    B, S, D = x.shape
    T = B * S
    N = T // BLOCK_T
    x2 = x.reshape(T, D)
    wt = W.T
    disp, probs, sel_t, wts_t = pl.pallas_call(
        _router_body,
        grid=(N,),
        in_specs=[
            pl.BlockSpec((BLOCK_T, D), lambda i: (i, 0)),
            pl.BlockSpec((D, NUM_EXPERTS), lambda i: (0, 0)),
        ],
        out_specs=[
            pl.BlockSpec((BLOCK_T, NUM_EXPERTS), lambda i: (i, 0)),
            pl.BlockSpec((BLOCK_T, NUM_EXPERTS), lambda i: (i, 0)),
            pl.BlockSpec((2, BLOCK_T), lambda i: (0, i)),
            pl.BlockSpec((2, BLOCK_T), lambda i: (0, i)),
        ],
        out_shape=[
            jax.ShapeDtypeStruct((T, NUM_EXPERTS), jnp.float32),
            jax.ShapeDtypeStruct((T, NUM_EXPERTS), jnp.float32),
            jax.ShapeDtypeStruct((2, T), jnp.int32),
            jax.ShapeDtypeStruct((2, T), jnp.float32),
        ],
    )(x2, wt)
    return (
        disp.reshape(B, S, NUM_EXPERTS),
        probs.reshape(B, S, NUM_EXPERTS),
        sel_t.T.reshape(B, S, 2),
        wts_t.T.reshape(B, S, 2),
    )
